# baseline (speedup 1.0000x reference)
import functools

import jax
import jax.numpy as jnp
import numpy as np
from jax import lax
from jax.experimental import pallas as pl
from jax.experimental.pallas import tpu as pltpu
from jax.experimental.pallas import tpu_sc as plsc


def _tc_transpose(dnn_t, BN=16384):
    D, NV = dnn_t.shape
    grid = (pl.cdiv(NV, BN),)

    def tr_k(in_ref, out_ref):
        out_ref[:] = in_ref[:].T

    return pl.pallas_call(
        tr_k,
        grid=grid,
        in_specs=[pl.BlockSpec((D, BN), lambda i: (0, i))],
        out_specs=pl.BlockSpec((BN, D), lambda i: (i, 0)),
        out_shape=jax.ShapeDtypeStruct((NV, D), jnp.float32),
    )(dnn_t)


def _sc_gather(xflat, pattern, linear_table, dnn_table, F, V, D):
    N = xflat.shape[0]
    info = plsc.get_sparse_core_info()
    NC, NS = info.num_cores, info.num_subcores
    NW = NC * NS
    PAT = pattern.shape[0]
    per_w = N // NW
    assert N % NW == 0 and per_w % PAT == 0
    GCH = 128
    SUB = F * GCH
    n_sub = per_w // SUB
    assert per_w % SUB == 0

    mesh = plsc.VectorSubcoreMesh(core_axis_name="c", subcore_axis_name="s")

    @functools.partial(
        pl.kernel,
        out_type=(
            jax.ShapeDtypeStruct((N, D), jnp.float32),
            jax.ShapeDtypeStruct((N,), jnp.float32),
        ),
        mesh=mesh,
        compiler_params=pltpu.CompilerParams(use_tc_tiling_on_sc=False),
        scratch_types=[
            pltpu.VMEM((per_w,), jnp.int32),
            pltpu.VMEM((per_w,), jnp.int32),
            pltpu.VMEM((SUB, D), jnp.float32),
            pltpu.VMEM((SUB,), jnp.float32),
            pltpu.VMEM((PAT,), jnp.int32),
            pltpu.SemaphoreType.DMA,
            pltpu.SemaphoreType.DMA,
        ],
    )
    def gather_k(x_hbm, pat_hbm, lt_hbm, dt_hbm, emb_out, lin_out,
                 x_v, idx_v, rows_v, lin_v, pat_v, sem_r, sem_l):
        wid = lax.axis_index("s") * NC + lax.axis_index("c")
        base = wid * per_w
        pltpu.sync_copy(x_hbm.at[pl.ds(base, per_w)], x_v)
        pltpu.sync_copy(pat_hbm, pat_v)

        n_groups = per_w // PAT
        chunks_per_group = PAT // 16

        def grp(g, carry):
            goff = pl.multiple_of(g * PAT, PAT)
            for t in range(chunks_per_group):
                o = goff + t * 16
                idx_v[pl.ds(o, 16)] = x_v[pl.ds(o, 16)] + pat_v[pl.ds(t * 16, 16)]
            return carry

        lax.fori_loop(0, n_groups, grp, 0)

        def step(s, carry):
            ib = pl.multiple_of(s * SUB, SUB)
            copies = []
            for j in range(F):
                o = ib + j * GCH
                copies.append(pltpu.async_copy(
                    dt_hbm.at[idx_v.at[pl.ds(o, GCH)]],
                    rows_v.at[pl.ds(j * GCH, GCH)], sem_r))
                copies.append(pltpu.async_copy(
                    lt_hbm.at[idx_v.at[pl.ds(o, GCH)]],
                    lin_v.at[pl.ds(j * GCH, GCH)], sem_l))
            for c in copies:
                c.wait()
            pltpu.sync_copy(rows_v, emb_out.at[pl.ds(base + ib, SUB)])
            pltpu.sync_copy(lin_v, lin_out.at[pl.ds(base + ib, SUB)])
            return carry

        lax.fori_loop(0, n_sub, step, 0)

    return gather_k(xflat, pattern, linear_table, dnn_table)


def _tc_mlp(emb, lin, W1, b1, W2, b2, W3, b3):
    B, FD = emb.shape
    F = lin.shape[1]
    BT = 2048
    assert B % BT == 0

    def mlp_k(emb_ref, lin_ref, W1_ref, b1_ref, W2_ref, b2_ref,
              W3_ref, b3_ref, out_ref):
        h = jnp.dot(emb_ref[:], W1_ref[:], preferred_element_type=jnp.float32)
        h = jnp.maximum(h + b1_ref[:], 0.0)
        h = jnp.dot(h, W2_ref[:], preferred_element_type=jnp.float32)
        h = jnp.maximum(h + b2_ref[:], 0.0)
        logit = jnp.dot(h, W3_ref[:], preferred_element_type=jnp.float32)
        logit = logit + b3_ref[:]
        logit = logit + jnp.sum(lin_ref[:], axis=1, keepdims=True)
        out_ref[:] = jax.nn.sigmoid(logit)

    grid = (B // BT,)
    return pl.pallas_call(
        mlp_k,
        grid=grid,
        in_specs=[
            pl.BlockSpec((BT, FD), lambda i: (i, 0)),
            pl.BlockSpec((BT, F), lambda i: (i, 0)),
            pl.BlockSpec(W1.shape, lambda i: (0, 0)),
            pl.BlockSpec(b1.shape, lambda i: (0, 0)),
            pl.BlockSpec(W2.shape, lambda i: (0, 0)),
            pl.BlockSpec(b2.shape, lambda i: (0, 0)),
            pl.BlockSpec(W3.shape, lambda i: (0, 0)),
            pl.BlockSpec(b3.shape, lambda i: (0, 0)),
        ],
        out_specs=pl.BlockSpec((BT, 1), lambda i: (i, 0)),
        out_shape=jax.ShapeDtypeStruct((B, 1), jnp.float32),
    )(emb, lin, W1, b1, W2, b2, W3, b3)


def kernel(X, linear_table, dnn_table, W1, b1, W2, b2, W3, b3):
    B, F = X.shape
    D = dnn_table.shape[1]
    V = linear_table.shape[0] // F
    PAT = int(np.lcm(16, F))
    pattern = jnp.asarray((np.arange(PAT) % F) * V, dtype=jnp.int32)
    xflat = X.reshape(-1)
    dnn_rm = _tc_transpose(dnn_table.T)
    emb_flat, lin_flat = _sc_gather(xflat, pattern, linear_table, dnn_rm,
                                    F, V, D)
    emb = emb_flat.reshape(B, F * D)
    lin = lin_flat.reshape(B, F)
    return _tc_mlp(emb, lin,
                   W1, b1.reshape(1, -1),
                   W2, b2.reshape(1, -1),
                   W3, b3.reshape(1, -1))


# speedup vs baseline: 1.1163x; 1.1163x over previous
import functools

import jax
import jax.numpy as jnp
import numpy as np
from jax import lax
from jax.experimental import pallas as pl
from jax.experimental.pallas import tpu as pltpu
from jax.experimental.pallas import tpu_sc as plsc


def _tc_transpose(dnn_t, BN=16384):
    D, NV = dnn_t.shape
    grid = (pl.cdiv(NV, BN),)

    def tr_k(in_ref, out_ref):
        out_ref[:] = in_ref[:].T.reshape(BN * D // 128, 128)

    return pl.pallas_call(
        tr_k,
        grid=grid,
        in_specs=[pl.BlockSpec((D, BN), lambda i: (0, i))],
        out_specs=pl.BlockSpec((BN * D // 128, 128), lambda i: (i, 0)),
        out_shape=jax.ShapeDtypeStruct((NV * D // 128, 128), jnp.float32),
    )(dnn_t)


def _sc_gather(xflat, pattern, linear_table, dnn_table, F, V, D):
    N = xflat.shape[0]
    info = plsc.get_sparse_core_info()
    NC, NS = info.num_cores, info.num_subcores
    NW = NC * NS
    PAT = pattern.shape[0]
    per_w = N // NW
    assert N % NW == 0 and per_w % PAT == 0
    GCH = 128
    SUB = F * GCH
    n_sub = per_w // SUB
    assert per_w % SUB == 0

    mesh = plsc.VectorSubcoreMesh(core_axis_name="c", subcore_axis_name="s")

    @functools.partial(
        pl.kernel,
        out_type=(
            jax.ShapeDtypeStruct((N, D), jnp.float32),
            jax.ShapeDtypeStruct((N,), jnp.float32),
        ),
        mesh=mesh,
        compiler_params=pltpu.CompilerParams(use_tc_tiling_on_sc=False),
        scratch_types=[
            pltpu.VMEM((per_w,), jnp.int32),
            pltpu.VMEM((per_w,), jnp.int32),
            pltpu.VMEM((SUB, D), jnp.float32),
            pltpu.VMEM((SUB,), jnp.float32),
            pltpu.VMEM((PAT,), jnp.int32),
            pltpu.SemaphoreType.DMA,
            pltpu.SemaphoreType.DMA,
        ],
    )
    def gather_k(x_hbm, pat_hbm, lt_hbm, dt_hbm, emb_out, lin_out,
                 x_v, idx_v, rows_v, lin_v, pat_v, sem_r, sem_l):
        wid = lax.axis_index("s") * NC + lax.axis_index("c")
        base = wid * per_w
        pltpu.sync_copy(x_hbm.at[pl.ds(base, per_w)], x_v)
        pltpu.sync_copy(pat_hbm, pat_v)

        n_groups = per_w // PAT
        chunks_per_group = PAT // 16

        def grp(g, carry):
            goff = pl.multiple_of(g * PAT, PAT)
            for t in range(chunks_per_group):
                o = goff + t * 16
                idx_v[pl.ds(o, 16)] = x_v[pl.ds(o, 16)] + pat_v[pl.ds(t * 16, 16)]
            return carry

        lax.fori_loop(0, n_groups, grp, 0)

        def step(s, carry):
            ib = pl.multiple_of(s * SUB, SUB)
            copies = []
            for j in range(F):
                o = ib + j * GCH
                copies.append(pltpu.async_copy(
                    dt_hbm.at[idx_v.at[pl.ds(o, GCH)]],
                    rows_v.at[pl.ds(j * GCH, GCH)], sem_r))
                copies.append(pltpu.async_copy(
                    lt_hbm.at[idx_v.at[pl.ds(o, GCH)]],
                    lin_v.at[pl.ds(j * GCH, GCH)], sem_l))
            for c in copies:
                c.wait()
            pltpu.sync_copy(rows_v, emb_out.at[pl.ds(base + ib, SUB)])
            pltpu.sync_copy(lin_v, lin_out.at[pl.ds(base + ib, SUB)])
            return carry

        lax.fori_loop(0, n_sub, step, 0)

    return gather_k(xflat, pattern, linear_table, dnn_table)


def _tc_mlp(emb, lin, W1, b1, W2, b2, W3, b3):
    B, FD = emb.shape
    F = lin.shape[1]
    BT = 2048
    assert B % BT == 0

    def mlp_k(emb_ref, lin_ref, W1_ref, b1_ref, W2_ref, b2_ref,
              W3_ref, b3_ref, out_ref):
        h = jnp.dot(emb_ref[:], W1_ref[:], preferred_element_type=jnp.float32)
        h = jnp.maximum(h + b1_ref[:], 0.0)
        h = jnp.dot(h, W2_ref[:], preferred_element_type=jnp.float32)
        h = jnp.maximum(h + b2_ref[:], 0.0)
        logit = jnp.dot(h, W3_ref[:], preferred_element_type=jnp.float32)
        logit = logit + b3_ref[:]
        logit = logit + jnp.sum(lin_ref[:], axis=1, keepdims=True)
        out_ref[:] = jax.nn.sigmoid(logit)

    grid = (B // BT,)
    return pl.pallas_call(
        mlp_k,
        grid=grid,
        in_specs=[
            pl.BlockSpec((BT, FD), lambda i: (i, 0)),
            pl.BlockSpec((BT, F), lambda i: (i, 0)),
            pl.BlockSpec(W1.shape, lambda i: (0, 0)),
            pl.BlockSpec(b1.shape, lambda i: (0, 0)),
            pl.BlockSpec(W2.shape, lambda i: (0, 0)),
            pl.BlockSpec(b2.shape, lambda i: (0, 0)),
            pl.BlockSpec(W3.shape, lambda i: (0, 0)),
            pl.BlockSpec(b3.shape, lambda i: (0, 0)),
        ],
        out_specs=pl.BlockSpec((BT, 1), lambda i: (i, 0)),
        out_shape=jax.ShapeDtypeStruct((B, 1), jnp.float32),
    )(emb, lin, W1, b1, W2, b2, W3, b3)


def kernel(X, linear_table, dnn_table, W1, b1, W2, b2, W3, b3):
    B, F = X.shape
    D = dnn_table.shape[1]
    V = linear_table.shape[0] // F
    PAT = int(np.lcm(16, F))
    pattern = jnp.asarray((np.arange(PAT) % F) * V, dtype=jnp.int32)
    xflat = X.reshape(-1)
    dnn_lin = jax.lax.optimization_barrier(dnn_table.reshape(-1))
    dnn_rm = dnn_lin.reshape(F * V, D)
    emb_flat, lin_flat = _sc_gather(xflat, pattern, linear_table, dnn_rm,
                                    F, V, D)
    emb = emb_flat.reshape(B, F * D)
    lin = lin_flat.reshape(B, F)
    return _tc_mlp(emb, lin,
                   W1, b1.reshape(1, -1),
                   W2, b2.reshape(1, -1),
                   W3, b3.reshape(1, -1))
